# Initial kernel scaffold; baseline (speedup 1.0000x reference)
import jax
import jax.numpy as jnp
from jax.experimental import pallas as pl


def kernel(graph_embedding, edge_index, e_feat, weight, hetero_prompt):
    raise NotImplementedError("write your pallas kernel here")


# speedup vs baseline: 2.8259x; 2.8259x over previous
import jax
import jax.numpy as jnp
from jax import lax
from jax.experimental import pallas as pl
from jax.experimental.pallas import tpu as pltpu
from jax.experimental.pallas import tpu_sc as plsc

N_NODES = 10000
N_EDGES = 320000
PD = 128
HD = PD // 2
DE = 16

NC = 2
NS = 16

E_PER_TILE = N_EDGES // NS
CHUNK = 80
N_CHUNKS = E_PER_TILE // CHUNK

NP = 10240
ROWS_PER_TILE = NP // NS
ROW_CHUNK = 128
N_ROW_CHUNKS = ROWS_PER_TILE // ROW_CHUNK


def _sc_body(emb_lo_hbm, emb_hi_hbm, src_hbm, dst_hbm, ef_hbm,
             o_emb_hbm, o_ef_hbm,
             idx_s, idx_d, rows, efrows, obuf, oebuf, acc_emb, acc_ef, sem):
    cid = lax.axis_index("c")
    sid = lax.axis_index("s")

    def zero_row(i, carry):
        for v in range(HD // 16):
            obuf[i, pl.ds(v * 16, 16)] = jnp.zeros((16,), jnp.float32)
        oebuf[i, pl.ds(0, 16)] = jnp.zeros((16,), jnp.float32)
        return carry

    lax.fori_loop(0, ROW_CHUNK, zero_row, 0)
    r0 = sid * ROWS_PER_TILE
    for rc in range(N_ROW_CHUNKS):
        pltpu.sync_copy(obuf, acc_emb.at[pl.ds(r0 + rc * ROW_CHUNK, ROW_CHUNK), :])
        pltpu.sync_copy(oebuf, acc_ef.at[pl.ds(r0 + rc * ROW_CHUNK, ROW_CHUNK), :])
    plsc.subcore_barrier()

    base0 = sid * E_PER_TILE

    def chunk_body(j, carry):
        base = pl.multiple_of(base0 + j * CHUNK, CHUNK)
        pltpu.sync_copy(src_hbm.at[pl.ds(base, CHUNK)], idx_s)
        pltpu.sync_copy(dst_hbm.at[pl.ds(base, CHUNK)], idx_d)

        @pl.when(cid == 0)
        def _():
            pltpu.async_copy(emb_lo_hbm.at[idx_s], rows, sem).wait()

        @pl.when(cid == 1)
        def _():
            pltpu.async_copy(emb_hi_hbm.at[idx_s], rows, sem).wait()
            pltpu.sync_copy(ef_hbm.at[pl.ds(base, CHUNK), :], efrows)
            pltpu.sync_copy(efrows, acc_ef.at[idx_d], add=True)

        pltpu.sync_copy(rows, acc_emb.at[idx_d], add=True)
        return carry

    lax.fori_loop(0, N_CHUNKS, chunk_body, 0)
    plsc.subcore_barrier()

    for rc in range(N_ROW_CHUNKS):
        rr = r0 + rc * ROW_CHUNK
        pltpu.sync_copy(acc_emb.at[pl.ds(rr, ROW_CHUNK), :], obuf)
        pltpu.sync_copy(obuf, o_emb_hbm.at[cid, pl.ds(rr, ROW_CHUNK), :])

        @pl.when(cid == 1)
        def _():
            pltpu.sync_copy(acc_ef.at[pl.ds(rr, ROW_CHUNK), :], oebuf)
            pltpu.sync_copy(oebuf, o_ef_hbm.at[pl.ds(rr, ROW_CHUNK), :])


_sc_scatter = pl.kernel(
    _sc_body,
    out_type=(
        jax.ShapeDtypeStruct((NC, NP, HD), jnp.float32),
        jax.ShapeDtypeStruct((NP, DE), jnp.float32),
    ),
    mesh=plsc.VectorSubcoreMesh(core_axis_name="c", subcore_axis_name="s"),
    scratch_types=[
        pltpu.VMEM((CHUNK,), jnp.int32),
        pltpu.VMEM((CHUNK,), jnp.int32),
        pltpu.VMEM((CHUNK, HD), jnp.float32),
        pltpu.VMEM((CHUNK, DE), jnp.float32),
        pltpu.VMEM((ROW_CHUNK, HD), jnp.float32),
        pltpu.VMEM((ROW_CHUNK, DE), jnp.float32),
        pltpu.VMEM_SHARED((NP, HD), jnp.float32),
        pltpu.VMEM_SHARED((NP, DE), jnp.float32),
        pltpu.SemaphoreType.DMA,
    ],
    compiler_params=pltpu.CompilerParams(use_tc_tiling_on_sc=False),
)


def _combine_body(pe_ref, pf_ref, w_ref, out_ref):
    emb = jnp.concatenate([pe_ref[0], pe_ref[1]], axis=1) * w_ref[0]
    out_ref[...] = jnp.concatenate([emb, pf_ref[...]], axis=1)


_ROWS_BLK = 2000


def _combine(o_emb, o_ef, weight):
    return pl.pallas_call(
        _combine_body,
        out_shape=jax.ShapeDtypeStruct((N_NODES, PD + DE), jnp.float32),
        grid=(N_NODES // _ROWS_BLK,),
        in_specs=[
            pl.BlockSpec((NC, _ROWS_BLK, HD), lambda i: (0, i, 0)),
            pl.BlockSpec((_ROWS_BLK, DE), lambda i: (i, 0)),
            pl.BlockSpec((1, PD), lambda i: (0, 0)),
        ],
        out_specs=pl.BlockSpec((_ROWS_BLK, PD + DE), lambda i: (i, 0)),
    )(o_emb, o_ef, weight)


def kernel(graph_embedding, edge_index, e_feat, weight, hetero_prompt):
    src = edge_index[0].astype(jnp.int32)
    dst = edge_index[1].astype(jnp.int32)
    emb_lo = graph_embedding[:, :HD]
    emb_hi = graph_embedding[:, HD:]
    o_emb, o_ef = _sc_scatter(emb_lo, emb_hi, src, dst, e_feat)
    return _combine(o_emb, o_ef, weight)


# speedup vs baseline: 6.7003x; 2.3710x over previous
import jax
import jax.numpy as jnp
from jax import lax
from jax.experimental import pallas as pl
from jax.experimental.pallas import tpu as pltpu
from jax.experimental.pallas import tpu_sc as plsc

N_NODES = 10000
N_EDGES = 320000
PD = 128
HD = PD // 2
DE = 16

NC = 2
NS = 16

E_PER_TILE = N_EDGES // NS
CHUNK = 80
N_CHUNKS = E_PER_TILE // CHUNK
HALF = N_CHUNKS // 2

NP = 10240
ROWS_PER_TILE = NP // NS
ROW_CHUNK = 128
N_ROW_CHUNKS = ROWS_PER_TILE // ROW_CHUNK


def _sc_body(emb2_hbm, src_hbm, dst_hbm, ef_hbm,
             o_emb_hbm, o_ef_hbm,
             idx_s, idx_d, rows0, rows1, efr0, efr1, obuf, oebuf,
             acc_emb, acc_ef, gsem0, gsem1, esem0, esem1):
    cid = lax.axis_index("c")
    sid = lax.axis_index("s")

    pltpu.sync_copy(src_hbm.at[sid], idx_s)
    pltpu.sync_copy(dst_hbm.at[sid], idx_d)

    def zero_row(i, carry):
        for v in range(HD // 16):
            obuf[i, pl.ds(v * 16, 16)] = jnp.zeros((16,), jnp.float32)
        oebuf[i, pl.ds(0, 16)] = jnp.zeros((16,), jnp.float32)
        return carry

    lax.fori_loop(0, ROW_CHUNK, zero_row, 0)
    r0 = sid * ROWS_PER_TILE
    for rc in range(N_ROW_CHUNKS):
        pltpu.sync_copy(obuf, acc_emb.at[pl.ds(r0 + rc * ROW_CHUNK, ROW_CHUNK), :])
        pltpu.sync_copy(oebuf, acc_ef.at[pl.ds(r0 + rc * ROW_CHUNK, ROW_CHUNK), :])
    plsc.subcore_barrier()

    base0 = sid * E_PER_TILE
    emb_t = emb2_hbm.at[cid]

    def ef_pred(j):
        return (j < HALF) != (cid == 1)

    def issue(j, rows_b, efr_b, gsem_b, esem_b):
        pltpu.async_copy(emb_t.at[idx_s.at[j]], rows_b, gsem_b)

        @pl.when(ef_pred(j))
        def _():
            base = pl.multiple_of(base0 + j * CHUNK, CHUNK)
            pltpu.async_copy(ef_hbm.at[pl.ds(base, CHUNK), :], efr_b, esem_b)

    def drain(j, rows_b, efr_b, gsem_b, esem_b):
        pltpu.make_async_copy(emb_t.at[pl.ds(0, CHUNK), :], rows_b, gsem_b).wait()
        pltpu.sync_copy(rows_b, acc_emb.at[idx_d.at[j]], add=True)

        @pl.when(ef_pred(j))
        def _():
            pltpu.make_async_copy(ef_hbm.at[pl.ds(0, CHUNK), :], efr_b, esem_b).wait()
            pltpu.sync_copy(efr_b, acc_ef.at[idx_d.at[j]], add=True)

    issue(0, rows0, efr0, gsem0, esem0)

    def body(i, carry):
        j0 = 2 * i
        j1 = j0 + 1
        issue(j1, rows1, efr1, gsem1, esem1)
        drain(j0, rows0, efr0, gsem0, esem0)

        @pl.when(j0 + 2 < N_CHUNKS)
        def _():
            issue(j0 + 2, rows0, efr0, gsem0, esem0)

        drain(j1, rows1, efr1, gsem1, esem1)
        return carry

    lax.fori_loop(0, N_CHUNKS // 2, body, 0)
    plsc.subcore_barrier()

    for rc in range(N_ROW_CHUNKS):
        rr = r0 + rc * ROW_CHUNK
        pltpu.sync_copy(acc_emb.at[pl.ds(rr, ROW_CHUNK), :], obuf)
        pltpu.sync_copy(obuf, o_emb_hbm.at[cid, pl.ds(rr, ROW_CHUNK), :])
        pltpu.sync_copy(acc_ef.at[pl.ds(rr, ROW_CHUNK), :], oebuf)
        pltpu.sync_copy(oebuf, o_ef_hbm.at[cid, pl.ds(rr, ROW_CHUNK), :])


_sc_scatter = pl.kernel(
    _sc_body,
    out_type=(
        jax.ShapeDtypeStruct((NC, NP, HD), jnp.float32),
        jax.ShapeDtypeStruct((NC, NP, DE), jnp.float32),
    ),
    mesh=plsc.VectorSubcoreMesh(core_axis_name="c", subcore_axis_name="s"),
    scratch_types=[
        pltpu.VMEM((N_CHUNKS, CHUNK), jnp.int32),
        pltpu.VMEM((N_CHUNKS, CHUNK), jnp.int32),
        pltpu.VMEM((CHUNK, HD), jnp.float32),
        pltpu.VMEM((CHUNK, HD), jnp.float32),
        pltpu.VMEM((CHUNK, DE), jnp.float32),
        pltpu.VMEM((CHUNK, DE), jnp.float32),
        pltpu.VMEM((ROW_CHUNK, HD), jnp.float32),
        pltpu.VMEM((ROW_CHUNK, DE), jnp.float32),
        pltpu.VMEM_SHARED((NP, HD), jnp.float32),
        pltpu.VMEM_SHARED((NP, DE), jnp.float32),
        pltpu.SemaphoreType.DMA,
        pltpu.SemaphoreType.DMA,
        pltpu.SemaphoreType.DMA,
        pltpu.SemaphoreType.DMA,
    ],
    compiler_params=pltpu.CompilerParams(use_tc_tiling_on_sc=False),
)


def _combine_body(pe_ref, pf_ref, w_ref, out_ref):
    emb = jnp.concatenate([pe_ref[0], pe_ref[1]], axis=1) * w_ref[0]
    out_ref[...] = jnp.concatenate([emb, pf_ref[0] + pf_ref[1]], axis=1)


_ROWS_BLK = 2000


def _combine(o_emb, o_ef, weight):
    return pl.pallas_call(
        _combine_body,
        out_shape=jax.ShapeDtypeStruct((N_NODES, PD + DE), jnp.float32),
        grid=(N_NODES // _ROWS_BLK,),
        in_specs=[
            pl.BlockSpec((NC, _ROWS_BLK, HD), lambda i: (0, i, 0)),
            pl.BlockSpec((NC, _ROWS_BLK, DE), lambda i: (0, i, 0)),
            pl.BlockSpec((1, PD), lambda i: (0, 0)),
        ],
        out_specs=pl.BlockSpec((_ROWS_BLK, PD + DE), lambda i: (i, 0)),
    )(o_emb, o_ef, weight)


def kernel(graph_embedding, edge_index, e_feat, weight, hetero_prompt):
    src = edge_index[0].astype(jnp.int32).reshape(NS, N_CHUNKS, CHUNK)
    dst = edge_index[1].astype(jnp.int32).reshape(NS, N_CHUNKS, CHUNK)
    emb2 = jnp.stack([graph_embedding[:, :HD], graph_embedding[:, HD:]], axis=0)
    o_emb, o_ef = _sc_scatter(emb2, src, dst, e_feat)
    return _combine(o_emb, o_ef, weight)


# speedup vs baseline: 7.3909x; 1.1031x over previous
import jax
import jax.numpy as jnp
from jax import lax
from jax.experimental import pallas as pl
from jax.experimental.pallas import tpu as pltpu
from jax.experimental.pallas import tpu_sc as plsc

N_NODES = 10000
N_EDGES = 320000
PD = 128
HD = PD // 2
DE = 16

NC = 2
NS = 16

E_PER_TILE = N_EDGES // NS
CHUNK = 80
N_CHUNKS = E_PER_TILE // CHUNK
HALF = N_CHUNKS // 2
NB = 5
NG = N_CHUNKS // NB
assert N_CHUNKS % NB == 0 and NG % 2 == 0

NP = 10000
ROWS_PER_TILE = NP // NS
ROW_CHUNK = 125
N_ROW_CHUNKS = ROWS_PER_TILE // ROW_CHUNK


def _sc_body(emb2_hbm, src_hbm, dst_hbm, ef_hbm, o_emb_hbm, o_ef_hbm, *scr):
    rows = scr[0:NB]
    efr = scr[NB:2 * NB]
    (ixs0, ixd0, ixs1, ixd1, obuf, oebuf, acc_emb, acc_ef) = scr[2 * NB:2 * NB + 8]
    base_sem = 2 * NB + 8
    gsem = scr[base_sem:base_sem + NB]
    esem = scr[base_sem + NB:base_sem + 2 * NB]
    isem0, isem1 = scr[base_sem + 2 * NB:base_sem + 2 * NB + 2]

    cid = lax.axis_index("c")
    sid = lax.axis_index("s")

    def zero_row(i, carry):
        for v in range(HD // 16):
            obuf[i, pl.ds(v * 16, 16)] = jnp.zeros((16,), jnp.float32)
        oebuf[i, pl.ds(0, 16)] = jnp.zeros((16,), jnp.float32)
        return carry

    lax.fori_loop(0, ROW_CHUNK, zero_row, 0)
    r0 = sid * ROWS_PER_TILE
    for rc in range(N_ROW_CHUNKS):
        pltpu.sync_copy(obuf, acc_emb.at[pl.ds(r0 + rc * ROW_CHUNK, ROW_CHUNK), :])
        pltpu.sync_copy(oebuf, acc_ef.at[pl.ds(r0 + rc * ROW_CHUNK, ROW_CHUNK), :])
    plsc.subcore_barrier()

    base0 = sid * E_PER_TILE
    emb_t = emb2_hbm.at[cid]

    def ef_pred(j):
        return (j < HALF) != (cid == 1)

    def gather_start(ixs, j, k):
        pltpu.async_copy(emb_t.at[ixs.at[k]], rows[k], gsem[k])

    def gather_wait(k):
        pltpu.make_async_copy(emb_t.at[pl.ds(0, CHUNK), :], rows[k], gsem[k]).wait()

    def ef_load_start(j, k):
        base = base0 + j * CHUNK
        pltpu.async_copy(ef_hbm.at[pl.ds(base, CHUNK), :], efr[k], esem[k])

    def ef_load_wait(k):
        pltpu.make_async_copy(ef_hbm.at[pl.ds(0, CHUNK), :], efr[k], esem[k]).wait()

    def idx_load_start(ixs, ixd, g, sem):
        pltpu.async_copy(src_hbm.at[sid, g], ixs, sem)
        pltpu.async_copy(dst_hbm.at[sid, g], ixd, sem)

    def idx_load_wait(ixs, ixd, sem):
        pltpu.make_async_copy(src_hbm.at[sid, 0], ixs, sem).wait()
        pltpu.make_async_copy(dst_hbm.at[sid, 0], ixd, sem).wait()

    def process_group(g, ixs, ixd, ixs_n, gather_next):
        for k in range(NB):
            j = g * NB + k
            gather_wait(k)
            pltpu.sync_copy(rows[k], acc_emb.at[ixd.at[k]], add=True)

            @pl.when(ef_pred(j))
            def _(j=j, k=k):
                ef_load_wait(k)
                pltpu.sync_copy(efr[k], acc_ef.at[ixd.at[k]], add=True)

            jn = j + NB

            @pl.when(gather_next)
            def _(jn=jn, k=k):
                gather_start(ixs_n, jn, k)

                @pl.when(ef_pred(jn))
                def _(jn=jn, k=k):
                    ef_load_start(jn, k)

    pltpu.sync_copy(src_hbm.at[sid, 0], ixs0)
    pltpu.sync_copy(dst_hbm.at[sid, 0], ixd0)
    idx_load_start(ixs1, ixd1, 1, isem1)
    for k in range(NB):
        gather_start(ixs0, k, k)

        @pl.when(ef_pred(k))
        def _(k=k):
            ef_load_start(k, k)

    def body(t, carry):
        ga = 2 * t
        gb = ga + 1
        idx_load_wait(ixs1, ixd1, isem1)
        process_group(ga, ixs0, ixd0, ixs1, True)

        @pl.when(ga + 2 < NG)
        def _():
            idx_load_start(ixs0, ixd0, ga + 2, isem0)

        @pl.when(ga + 2 < NG)
        def _():
            idx_load_wait(ixs0, ixd0, isem0)

        process_group(gb, ixs1, ixd1, ixs0, gb + 1 < NG)

        @pl.when(gb + 2 < NG)
        def _():
            idx_load_start(ixs1, ixd1, gb + 2, isem1)

        return carry

    lax.fori_loop(0, NG // 2, body, 0)
    plsc.subcore_barrier()

    for rc in range(N_ROW_CHUNKS):
        rr = r0 + rc * ROW_CHUNK
        pltpu.sync_copy(acc_emb.at[pl.ds(rr, ROW_CHUNK), :], obuf)
        pltpu.sync_copy(obuf, o_emb_hbm.at[cid, pl.ds(rr, ROW_CHUNK), :])
        pltpu.sync_copy(acc_ef.at[pl.ds(rr, ROW_CHUNK), :], oebuf)
        pltpu.sync_copy(oebuf, o_ef_hbm.at[cid, pl.ds(rr, ROW_CHUNK), :])


_sc_scatter = pl.kernel(
    _sc_body,
    out_type=(
        jax.ShapeDtypeStruct((NC, NP, HD), jnp.float32),
        jax.ShapeDtypeStruct((NC, NP, DE), jnp.float32),
    ),
    mesh=plsc.VectorSubcoreMesh(core_axis_name="c", subcore_axis_name="s"),
    scratch_types=(
        [pltpu.VMEM((CHUNK, HD), jnp.float32) for _ in range(NB)]
        + [pltpu.VMEM((CHUNK, DE), jnp.float32) for _ in range(NB)]
        + [
            pltpu.VMEM((NB, CHUNK), jnp.int32),
            pltpu.VMEM((NB, CHUNK), jnp.int32),
            pltpu.VMEM((NB, CHUNK), jnp.int32),
            pltpu.VMEM((NB, CHUNK), jnp.int32),
            pltpu.VMEM((ROW_CHUNK, HD), jnp.float32),
            pltpu.VMEM((ROW_CHUNK, DE), jnp.float32),
            pltpu.VMEM_SHARED((NP, HD), jnp.float32),
            pltpu.VMEM_SHARED((NP, DE), jnp.float32),
        ]
        + [pltpu.SemaphoreType.DMA for _ in range(2 * NB + 2)]
    ),
    compiler_params=pltpu.CompilerParams(use_tc_tiling_on_sc=False),
)


def _combine_body(pe_ref, pf_ref, w_ref, out_ref):
    emb = jnp.concatenate([pe_ref[0], pe_ref[1]], axis=1) * w_ref[0]
    out_ref[...] = jnp.concatenate([emb, pf_ref[0] + pf_ref[1]], axis=1)


_ROWS_BLK = 2000


def _combine(o_emb, o_ef, weight):
    return pl.pallas_call(
        _combine_body,
        out_shape=jax.ShapeDtypeStruct((N_NODES, PD + DE), jnp.float32),
        grid=(N_NODES // _ROWS_BLK,),
        in_specs=[
            pl.BlockSpec((NC, _ROWS_BLK, HD), lambda i: (0, i, 0)),
            pl.BlockSpec((NC, _ROWS_BLK, DE), lambda i: (0, i, 0)),
            pl.BlockSpec((1, PD), lambda i: (0, 0)),
        ],
        out_specs=pl.BlockSpec((_ROWS_BLK, PD + DE), lambda i: (i, 0)),
    )(o_emb, o_ef, weight)


def kernel(graph_embedding, edge_index, e_feat, weight, hetero_prompt):
    src = edge_index[0].astype(jnp.int32).reshape(NS, NG, NB, CHUNK)
    dst = edge_index[1].astype(jnp.int32).reshape(NS, NG, NB, CHUNK)
    emb2 = jnp.stack([graph_embedding[:, :HD], graph_embedding[:, HD:]], axis=0)
    o_emb, o_ef = _sc_scatter(emb2, src, dst, e_feat)
    return _combine(o_emb, o_ef, weight)


# speedup vs baseline: 9.2943x; 1.2575x over previous
import jax
import jax.numpy as jnp
from jax import lax
from jax.experimental import pallas as pl
from jax.experimental.pallas import tpu as pltpu
from jax.experimental.pallas import tpu_sc as plsc

N_NODES = 10000
N_EDGES = 320000
PD = 128
HD = PD // 2
DE = 16

NC = 2
NS = 16
NW = NC * NS

CHUNK = 80
NB = 5

E_PER_TILE = N_EDGES // NS
N_CHUNKS = E_PER_TILE // CHUNK
NG = N_CHUNKS // NB
assert N_CHUNKS % NB == 0 and NG % 2 == 0

EB_PER_TILE = N_EDGES // NW
NB_CHUNKS = EB_PER_TILE // CHUNK
NGB = NB_CHUNKS // NB
assert NB_CHUNKS % NB == 0

NP = 10000
ROWS_PER_TILE = NP // NS
ROW_CHUNK = 125
N_ROW_CHUNKS = ROWS_PER_TILE // ROW_CHUNK


def _emb_body(emb2_hbm, src_hbm, dst_hbm, o_emb_hbm, *scr):
    rows = scr[0:NB]
    ixs0, ixd0, ixs1, ixd1, obuf, acc_emb = scr[NB:NB + 6]
    gsem = scr[NB + 6:NB + 6 + NB]
    isem0, isem1 = scr[NB + 6 + NB:NB + 6 + NB + 2]

    cid = lax.axis_index("c")
    sid = lax.axis_index("s")

    def zero_row(i, carry):
        for v in range(HD // 16):
            obuf[i, pl.ds(v * 16, 16)] = jnp.zeros((16,), jnp.float32)
        return carry

    lax.fori_loop(0, ROW_CHUNK, zero_row, 0)
    r0 = sid * ROWS_PER_TILE
    for rc in range(N_ROW_CHUNKS):
        pltpu.sync_copy(obuf, acc_emb.at[pl.ds(r0 + rc * ROW_CHUNK, ROW_CHUNK), :])
    plsc.subcore_barrier()

    emb_t = emb2_hbm.at[cid]

    def gather_start(ixs, k):
        pltpu.async_copy(emb_t.at[ixs.at[k]], rows[k], gsem[k])

    def gather_wait(k):
        pltpu.make_async_copy(emb_t.at[pl.ds(0, CHUNK), :], rows[k], gsem[k]).wait()

    def idx_load_start(ixs, ixd, g, sem):
        pltpu.async_copy(src_hbm.at[sid, g], ixs, sem)
        pltpu.async_copy(dst_hbm.at[sid, g], ixd, sem)

    def idx_load_wait(ixs, ixd, sem):
        pltpu.make_async_copy(src_hbm.at[sid, 0], ixs, sem).wait()
        pltpu.make_async_copy(dst_hbm.at[sid, 0], ixd, sem).wait()

    def process_group(ixd, ixs_n, gather_next):
        for k in range(NB):
            gather_wait(k)
            pltpu.sync_copy(rows[k], acc_emb.at[ixd.at[k]], add=True)

            @pl.when(gather_next)
            def _(k=k):
                gather_start(ixs_n, k)

    pltpu.sync_copy(src_hbm.at[sid, 0], ixs0)
    pltpu.sync_copy(dst_hbm.at[sid, 0], ixd0)
    idx_load_start(ixs1, ixd1, 1, isem1)
    for k in range(NB):
        gather_start(ixs0, k)

    def body(t, carry):
        ga = 2 * t
        idx_load_wait(ixs1, ixd1, isem1)
        process_group(ixd0, ixs1, True)

        @pl.when(ga + 2 < NG)
        def _():
            idx_load_start(ixs0, ixd0, ga + 2, isem0)
            idx_load_wait(ixs0, ixd0, isem0)

        process_group(ixd1, ixs0, ga + 2 < NG)

        @pl.when(ga + 3 < NG)
        def _():
            idx_load_start(ixs1, ixd1, ga + 3, isem1)

        return carry

    lax.fori_loop(0, NG // 2, body, 0)
    plsc.subcore_barrier()

    for rc in range(N_ROW_CHUNKS):
        rr = r0 + rc * ROW_CHUNK
        pltpu.sync_copy(acc_emb.at[pl.ds(rr, ROW_CHUNK), :], obuf)
        pltpu.sync_copy(obuf, o_emb_hbm.at[cid, pl.ds(rr, ROW_CHUNK), :])


_emb_scatter = pl.kernel(
    _emb_body,
    out_type=jax.ShapeDtypeStruct((NC, NP, HD), jnp.float32),
    mesh=plsc.VectorSubcoreMesh(core_axis_name="c", subcore_axis_name="s"),
    scratch_types=(
        [pltpu.VMEM((CHUNK, HD), jnp.float32) for _ in range(NB)]
        + [
            pltpu.VMEM((NB, CHUNK), jnp.int32),
            pltpu.VMEM((NB, CHUNK), jnp.int32),
            pltpu.VMEM((NB, CHUNK), jnp.int32),
            pltpu.VMEM((NB, CHUNK), jnp.int32),
            pltpu.VMEM((ROW_CHUNK, HD), jnp.float32),
            pltpu.VMEM_SHARED((NP, HD), jnp.float32),
        ]
        + [pltpu.SemaphoreType.DMA for _ in range(NB + 2)]
    ),
    compiler_params=pltpu.CompilerParams(use_tc_tiling_on_sc=False),
)


def _ef_body(ef_hbm, dstb_hbm, o_ef_hbm, *scr):
    efr = scr[0:NB]
    ixd0, ixd1, oebuf, acc_ef = scr[NB:NB + 4]
    esem = scr[NB + 4:NB + 4 + NB]
    isem0, isem1 = scr[NB + 4 + NB:NB + 4 + NB + 2]

    cid = lax.axis_index("c")
    sid = lax.axis_index("s")
    wid = cid * NS + sid

    def zero_row(i, carry):
        oebuf[i, pl.ds(0, 16)] = jnp.zeros((16,), jnp.float32)
        return carry

    lax.fori_loop(0, ROW_CHUNK, zero_row, 0)
    r0 = sid * ROWS_PER_TILE
    for rc in range(N_ROW_CHUNKS):
        pltpu.sync_copy(oebuf, acc_ef.at[pl.ds(r0 + rc * ROW_CHUNK, ROW_CHUNK), :])
    plsc.subcore_barrier()

    base0 = wid * EB_PER_TILE

    def ef_load_start(j, k):
        base = base0 + j * CHUNK
        pltpu.async_copy(ef_hbm.at[pl.ds(base, CHUNK), :], efr[k], esem[k])

    def ef_load_wait(k):
        pltpu.make_async_copy(ef_hbm.at[pl.ds(0, CHUNK), :], efr[k], esem[k]).wait()

    def idx_load_start(ixd, g, sem):
        pltpu.async_copy(dstb_hbm.at[wid, g], ixd, sem)

    def idx_load_wait(ixd, sem):
        pltpu.make_async_copy(dstb_hbm.at[wid, 0], ixd, sem).wait()

    def process_group(g, ixd, load_next):
        for k in range(NB):
            ef_load_wait(k)
            pltpu.sync_copy(efr[k], acc_ef.at[ixd.at[k]], add=True)

            @pl.when(load_next)
            def _(g=g, k=k):
                ef_load_start((g + 1) * NB + k, k)

    pltpu.sync_copy(dstb_hbm.at[wid, 0], ixd0)
    idx_load_start(ixd1, 1, isem1)
    for k in range(NB):
        ef_load_start(k, k)

    def body(t, carry):
        ga = 2 * t
        idx_load_wait(ixd1, isem1)
        process_group(ga, ixd0, True)

        @pl.when(ga + 2 < NGB)
        def _():
            idx_load_start(ixd0, ga + 2, isem0)
            idx_load_wait(ixd0, isem0)

        process_group(ga + 1, ixd1, ga + 2 < NGB)

        @pl.when(ga + 3 < NGB)
        def _():
            idx_load_start(ixd1, ga + 3, isem1)

        return carry

    lax.fori_loop(0, NGB // 2, body, 0)
    process_group(NGB - 1, ixd0, False)
    plsc.subcore_barrier()

    for rc in range(N_ROW_CHUNKS):
        rr = r0 + rc * ROW_CHUNK
        pltpu.sync_copy(acc_ef.at[pl.ds(rr, ROW_CHUNK), :], oebuf)
        pltpu.sync_copy(oebuf, o_ef_hbm.at[cid, pl.ds(rr, ROW_CHUNK), :])


_ef_scatter = pl.kernel(
    _ef_body,
    out_type=jax.ShapeDtypeStruct((NC, NP, DE), jnp.float32),
    mesh=plsc.VectorSubcoreMesh(core_axis_name="c", subcore_axis_name="s"),
    scratch_types=(
        [pltpu.VMEM((CHUNK, DE), jnp.float32) for _ in range(NB)]
        + [
            pltpu.VMEM((NB, CHUNK), jnp.int32),
            pltpu.VMEM((NB, CHUNK), jnp.int32),
            pltpu.VMEM((ROW_CHUNK, DE), jnp.float32),
            pltpu.VMEM_SHARED((NP, DE), jnp.float32),
        ]
        + [pltpu.SemaphoreType.DMA for _ in range(NB + 2)]
    ),
    compiler_params=pltpu.CompilerParams(use_tc_tiling_on_sc=False),
)


def _combine_body(pe_ref, pf_ref, w_ref, out_ref):
    emb = jnp.concatenate([pe_ref[0], pe_ref[1]], axis=1) * w_ref[0]
    out_ref[...] = jnp.concatenate([emb, pf_ref[0] + pf_ref[1]], axis=1)


_ROWS_BLK = 2000


def _combine(o_emb, o_ef, weight):
    return pl.pallas_call(
        _combine_body,
        out_shape=jax.ShapeDtypeStruct((N_NODES, PD + DE), jnp.float32),
        grid=(N_NODES // _ROWS_BLK,),
        in_specs=[
            pl.BlockSpec((NC, _ROWS_BLK, HD), lambda i: (0, i, 0)),
            pl.BlockSpec((NC, _ROWS_BLK, DE), lambda i: (0, i, 0)),
            pl.BlockSpec((1, PD), lambda i: (0, 0)),
        ],
        out_specs=pl.BlockSpec((_ROWS_BLK, PD + DE), lambda i: (i, 0)),
    )(o_emb, o_ef, weight)


def kernel(graph_embedding, edge_index, e_feat, weight, hetero_prompt):
    src32 = edge_index[0].astype(jnp.int32)
    dst32 = edge_index[1].astype(jnp.int32)
    src = src32.reshape(NS, NG, NB, CHUNK)
    dst = dst32.reshape(NS, NG, NB, CHUNK)
    dstb = dst32.reshape(NW, NGB, NB, CHUNK)
    emb2 = jnp.stack([graph_embedding[:, :HD], graph_embedding[:, HD:]], axis=0)
    o_emb = _emb_scatter(emb2, src, dst)
    o_emb, dstb = lax.optimization_barrier((o_emb, dstb))
    o_ef = _ef_scatter(e_feat, dstb)
    return _combine(o_emb, o_ef, weight)


# speedup vs baseline: 9.8566x; 1.0605x over previous
import jax
import jax.numpy as jnp
from jax import lax
from jax.experimental import pallas as pl
from jax.experimental.pallas import tpu as pltpu
from jax.experimental.pallas import tpu_sc as plsc

N_NODES = 10000
N_EDGES = 320000
PD = 128
HD = PD // 2
DE = 16

NC = 2
NS = 16
NW = NC * NS

CHUNK = 80
NB = 5

E_PER_TILE = N_EDGES // NS
N_CHUNKS = E_PER_TILE // CHUNK
NG = N_CHUNKS // NB
assert N_CHUNKS % NB == 0 and NG % 2 == 0

EB_PER_TILE = N_EDGES // NW
NB_CHUNKS = EB_PER_TILE // CHUNK
NGB = NB_CHUNKS // NB
assert NB_CHUNKS % NB == 0

NP = 10000
ROWS_PER_TILE = NP // NS
ROW_CHUNK = 125
N_ROW_CHUNKS = ROWS_PER_TILE // ROW_CHUNK


def _emb_body(embr_hbm, src_hbm, dst_hbm, o_emb_hbm, *scr):
    rows = scr[0:NB]
    ixs0, ixd0, ixs1, ixd1, ixt0, ixt1, obuf, acc_emb = scr[NB:NB + 8]
    gsem = scr[NB + 8:NB + 8 + NB]
    isem0, isem1 = scr[NB + 8 + NB:NB + 8 + NB + 2]

    cid = lax.axis_index("c")
    sid = lax.axis_index("s")

    def zero_row(i, carry):
        for v in range(HD // 16):
            obuf[i, pl.ds(v * 16, 16)] = jnp.zeros((16,), jnp.float32)
        return carry

    lax.fori_loop(0, ROW_CHUNK, zero_row, 0)
    r0 = sid * ROWS_PER_TILE
    for rc in range(N_ROW_CHUNKS):
        pltpu.sync_copy(obuf, acc_emb.at[pl.ds(r0 + rc * ROW_CHUNK, ROW_CHUNK), :])
    plsc.subcore_barrier()

    def transform(ixs, ixt):
        for k in range(NB):
            for v in range(CHUNK // 16):
                s = ixs[k, pl.ds(v * 16, 16)]
                ixt[k, pl.ds(v * 16, 16)] = s * 2 + cid

    def gather_start(ixt, k):
        pltpu.async_copy(embr_hbm.at[ixt.at[k]], rows[k], gsem[k])

    def gather_wait(k):
        pltpu.make_async_copy(embr_hbm.at[pl.ds(0, CHUNK), :], rows[k], gsem[k]).wait()

    def idx_load_start(ixs, ixd, g, sem):
        pltpu.async_copy(src_hbm.at[sid, g], ixs, sem)
        pltpu.async_copy(dst_hbm.at[sid, g], ixd, sem)

    def idx_load_wait(ixs, ixd, sem):
        pltpu.make_async_copy(src_hbm.at[sid, 0], ixs, sem).wait()
        pltpu.make_async_copy(dst_hbm.at[sid, 0], ixd, sem).wait()

    def process_group(ixd, ixt_n, gather_next):
        for k in range(NB):
            gather_wait(k)
            pltpu.sync_copy(rows[k], acc_emb.at[ixd.at[k]], add=True)

            @pl.when(gather_next)
            def _(k=k):
                gather_start(ixt_n, k)

    pltpu.sync_copy(src_hbm.at[sid, 0], ixs0)
    pltpu.sync_copy(dst_hbm.at[sid, 0], ixd0)
    idx_load_start(ixs1, ixd1, 1, isem1)
    transform(ixs0, ixt0)
    for k in range(NB):
        gather_start(ixt0, k)

    def body(t, carry):
        ga = 2 * t
        idx_load_wait(ixs1, ixd1, isem1)
        transform(ixs1, ixt1)
        process_group(ixd0, ixt1, True)

        @pl.when(ga + 2 < NG)
        def _():
            idx_load_start(ixs0, ixd0, ga + 2, isem0)
            idx_load_wait(ixs0, ixd0, isem0)
            transform(ixs0, ixt0)

        process_group(ixd1, ixt0, ga + 2 < NG)

        @pl.when(ga + 3 < NG)
        def _():
            idx_load_start(ixs1, ixd1, ga + 3, isem1)

        return carry

    lax.fori_loop(0, NG // 2, body, 0)
    plsc.subcore_barrier()

    for rc in range(N_ROW_CHUNKS):
        rr = r0 + rc * ROW_CHUNK
        pltpu.sync_copy(acc_emb.at[pl.ds(rr, ROW_CHUNK), :], obuf)
        pltpu.sync_copy(obuf, o_emb_hbm.at[cid, pl.ds(rr, ROW_CHUNK), :])


_emb_scatter = pl.kernel(
    _emb_body,
    out_type=jax.ShapeDtypeStruct((NC, NP, HD), jnp.float32),
    mesh=plsc.VectorSubcoreMesh(core_axis_name="c", subcore_axis_name="s"),
    scratch_types=(
        [pltpu.VMEM((CHUNK, HD), jnp.float32) for _ in range(NB)]
        + [pltpu.VMEM((NB, CHUNK), jnp.int32) for _ in range(6)]
        + [
            pltpu.VMEM((ROW_CHUNK, HD), jnp.float32),
            pltpu.VMEM_SHARED((NP, HD), jnp.float32),
        ]
        + [pltpu.SemaphoreType.DMA for _ in range(NB + 2)]
    ),
    compiler_params=pltpu.CompilerParams(use_tc_tiling_on_sc=False),
)


def _ef_body(ef_hbm, dstb_hbm, o_ef_hbm, *scr):
    efr = scr[0:NB]
    ixd0, ixd1, oebuf, acc_ef = scr[NB:NB + 4]
    esem = scr[NB + 4:NB + 4 + NB]
    isem0, isem1 = scr[NB + 4 + NB:NB + 4 + NB + 2]

    cid = lax.axis_index("c")
    sid = lax.axis_index("s")
    wid = cid * NS + sid

    def zero_row(i, carry):
        oebuf[i, pl.ds(0, 16)] = jnp.zeros((16,), jnp.float32)
        return carry

    lax.fori_loop(0, ROW_CHUNK, zero_row, 0)
    r0 = sid * ROWS_PER_TILE
    for rc in range(N_ROW_CHUNKS):
        pltpu.sync_copy(oebuf, acc_ef.at[pl.ds(r0 + rc * ROW_CHUNK, ROW_CHUNK), :])
    plsc.subcore_barrier()

    base0 = wid * EB_PER_TILE

    def ef_load_start(j, k):
        base = base0 + j * CHUNK
        pltpu.async_copy(ef_hbm.at[pl.ds(base, CHUNK), :], efr[k], esem[k])

    def ef_load_wait(k):
        pltpu.make_async_copy(ef_hbm.at[pl.ds(0, CHUNK), :], efr[k], esem[k]).wait()

    def idx_load_start(ixd, g, sem):
        pltpu.async_copy(dstb_hbm.at[wid, g], ixd, sem)

    def idx_load_wait(ixd, sem):
        pltpu.make_async_copy(dstb_hbm.at[wid, 0], ixd, sem).wait()

    def process_group(g, ixd, load_next):
        for k in range(NB):
            ef_load_wait(k)
            pltpu.sync_copy(efr[k], acc_ef.at[ixd.at[k]], add=True)

            @pl.when(load_next)
            def _(g=g, k=k):
                ef_load_start((g + 1) * NB + k, k)

    pltpu.sync_copy(dstb_hbm.at[wid, 0], ixd0)
    idx_load_start(ixd1, 1, isem1)
    for k in range(NB):
        ef_load_start(k, k)

    def body(t, carry):
        ga = 2 * t
        idx_load_wait(ixd1, isem1)
        process_group(ga, ixd0, True)

        @pl.when(ga + 2 < NGB)
        def _():
            idx_load_start(ixd0, ga + 2, isem0)
            idx_load_wait(ixd0, isem0)

        process_group(ga + 1, ixd1, ga + 2 < NGB)

        @pl.when(ga + 3 < NGB)
        def _():
            idx_load_start(ixd1, ga + 3, isem1)

        return carry

    lax.fori_loop(0, NGB // 2, body, 0)
    process_group(NGB - 1, ixd0, False)
    plsc.subcore_barrier()

    for rc in range(N_ROW_CHUNKS):
        rr = r0 + rc * ROW_CHUNK
        pltpu.sync_copy(acc_ef.at[pl.ds(rr, ROW_CHUNK), :], oebuf)
        pltpu.sync_copy(oebuf, o_ef_hbm.at[cid, pl.ds(rr, ROW_CHUNK), :])


_ef_scatter = pl.kernel(
    _ef_body,
    out_type=jax.ShapeDtypeStruct((NC, NP, DE), jnp.float32),
    mesh=plsc.VectorSubcoreMesh(core_axis_name="c", subcore_axis_name="s"),
    scratch_types=(
        [pltpu.VMEM((CHUNK, DE), jnp.float32) for _ in range(NB)]
        + [
            pltpu.VMEM((NB, CHUNK), jnp.int32),
            pltpu.VMEM((NB, CHUNK), jnp.int32),
            pltpu.VMEM((ROW_CHUNK, DE), jnp.float32),
            pltpu.VMEM_SHARED((NP, DE), jnp.float32),
        ]
        + [pltpu.SemaphoreType.DMA for _ in range(NB + 2)]
    ),
    compiler_params=pltpu.CompilerParams(use_tc_tiling_on_sc=False),
)


def _combine_body(pe_ref, pf_ref, w_ref, out_ref):
    emb = jnp.concatenate([pe_ref[0], pe_ref[1]], axis=1) * w_ref[0]
    out_ref[...] = jnp.concatenate([emb, pf_ref[0] + pf_ref[1]], axis=1)


_ROWS_BLK = 2000


def _combine(o_emb, o_ef, weight):
    return pl.pallas_call(
        _combine_body,
        out_shape=jax.ShapeDtypeStruct((N_NODES, PD + DE), jnp.float32),
        grid=(N_NODES // _ROWS_BLK,),
        in_specs=[
            pl.BlockSpec((NC, _ROWS_BLK, HD), lambda i: (0, i, 0)),
            pl.BlockSpec((NC, _ROWS_BLK, DE), lambda i: (0, i, 0)),
            pl.BlockSpec((1, PD), lambda i: (0, 0)),
        ],
        out_specs=pl.BlockSpec((_ROWS_BLK, PD + DE), lambda i: (i, 0)),
    )(o_emb, o_ef, weight)


def kernel(graph_embedding, edge_index, e_feat, weight, hetero_prompt):
    src32 = edge_index[0].astype(jnp.int32)
    dst32 = edge_index[1].astype(jnp.int32)
    src = src32.reshape(NS, NG, NB, CHUNK)
    dst = dst32.reshape(NS, NG, NB, CHUNK)
    dstb = dst32.reshape(NW, NGB, NB, CHUNK)
    embr = graph_embedding.reshape(2 * N_NODES, HD)
    o_emb = _emb_scatter(embr, src, dst)
    o_emb, dstb = lax.optimization_barrier((o_emb, dstb))
    o_ef = _ef_scatter(e_feat, dstb)
    return _combine(o_emb, o_ef, weight)
